# Initial kernel scaffold; baseline (speedup 1.0000x reference)
import jax
import jax.numpy as jnp
from jax.experimental import pallas as pl


def kernel(input, weight, grid):
    raise NotImplementedError("write your pallas kernel here")


# speedup vs baseline: 262.2730x; 262.2730x over previous
import functools

import jax
import jax.numpy as jnp
from jax.experimental import pallas as pl

_M0 = 0.05
_MAX_FINITE_BITS = 0x7F7FFFFF
_BISECT_ITERS = 31


def _maxk_kernel(w_ref, out_ref):
    w = w_ref[:, :]
    n = w.shape[1]
    wb = _M0 * jnp.sum(w, axis=1, keepdims=True)
    wi = jax.lax.bitcast_convert_type(w, jnp.int32)
    lo0 = jnp.full((w.shape[0], 1), -1, jnp.int32)
    hi0 = jnp.full((w.shape[0], 1), _MAX_FINITE_BITS, jnp.int32)

    def body(_, carry):
        lo, hi = carry
        mid = lo + ((hi - lo) >> 1)
        mask = wi <= mid
        s_le = jnp.sum(jnp.where(mask, w, 0.0), axis=1, keepdims=True)
        cond = s_le >= wb
        return jnp.where(cond, lo, mid), jnp.where(cond, mid, hi)

    _, hi = jax.lax.fori_loop(0, _BISECT_ITERS, body, (lo0, hi0))
    tau = jax.lax.bitcast_convert_type(hi, jnp.float32)
    mlt = wi < hi
    s_lt = jnp.sum(jnp.where(mlt, w, 0.0), axis=1, keepdims=True)
    c_lt = jnp.sum(mlt.astype(jnp.float32), axis=1, keepdims=True)
    safe_tau = jnp.where(tau > 0, tau, 1.0)
    extra = jnp.where(
        tau > 0, jnp.floor(jnp.maximum(wb - s_lt, 0.0) / safe_tau), 0.0
    )
    counts = c_lt + extra
    maxk = jnp.minimum(jnp.max(counts, axis=0, keepdims=True) + 1.0, float(n))
    out_ref[:, :] = maxk


def _dtm_kernel(x_ref, w_ref, gt_ref, mk_ref, out_ref):
    x = x_ref[0]
    gt = gt_ref[:, :]
    w = w_ref[0]
    mk = mk_ref[:, :]
    wb = _M0 * jnp.sum(w, axis=1, keepdims=True)

    x2 = jnp.sum(x * x, axis=1, keepdims=True)
    g2 = jnp.sum(gt * gt, axis=0, keepdims=True)
    xg = jnp.dot(x, gt, preferred_element_type=jnp.float32)
    s = jnp.maximum(x2 + g2 - 2.0 * xg, 0.0)
    si = jax.lax.bitcast_convert_type(s, jnp.int32)

    tr = s.shape[0]
    lo0 = jnp.full((tr, 1), -1, jnp.int32)
    hi0 = jnp.full((tr, 1), _MAX_FINITE_BITS, jnp.int32)

    def body(_, carry):
        lo, hi = carry
        mid = lo + ((hi - lo) >> 1)
        mask = si <= mid
        w_le = jnp.sum(jnp.where(mask, w, 0.0), axis=1, keepdims=True)
        c_le = jnp.sum(mask.astype(jnp.float32), axis=1, keepdims=True)
        cond = (w_le >= wb) | (c_le >= mk)
        return jnp.where(cond, lo, mid), jnp.where(cond, mid, hi)

    _, hi = jax.lax.fori_loop(0, _BISECT_ITERS, body, (lo0, hi0))
    s_star = jax.lax.bitcast_convert_type(hi, jnp.float32)
    mlt = si < hi
    w_lt = jnp.sum(jnp.where(mlt, w, 0.0), axis=1, keepdims=True)
    g_lt = jnp.sum(jnp.where(mlt, s * w, 0.0), axis=1, keepdims=True)
    val = g_lt + s_star * (wb - w_lt)
    out_ref[0] = jnp.sqrt(val / wb)


@jax.jit
def kernel(input, weight, grid):
    b, n, d = input.shape
    m = grid.shape[0]

    maxk = pl.pallas_call(
        _maxk_kernel,
        out_shape=jax.ShapeDtypeStruct((1, 1), jnp.float32),
    )(weight)

    tr = 256 if n % 256 == 0 else n
    gt = grid.T

    out = pl.pallas_call(
        _dtm_kernel,
        grid=(b, n // tr),
        in_specs=[
            pl.BlockSpec((1, tr, d), lambda bi, ti: (bi, ti, 0)),
            pl.BlockSpec((1, 1, m), lambda bi, ti: (bi, 0, 0)),
            pl.BlockSpec((d, m), lambda bi, ti: (0, 0)),
            pl.BlockSpec((1, 1), lambda bi, ti: (0, 0)),
        ],
        out_specs=pl.BlockSpec((1, tr, 1), lambda bi, ti: (bi, ti, 0)),
        out_shape=jax.ShapeDtypeStruct((b, n, 1), jnp.float32),
    )(input, weight.reshape(b, 1, n), gt, maxk)

    return out[:, :, 0]


# speedup vs baseline: 467.0992x; 1.7810x over previous
import functools

import jax
import jax.numpy as jnp
from jax.experimental import pallas as pl

_M0 = 0.05
_MAX_FINITE_BITS = 0x7F7FFFFF
_BISECT_ITERS = 31


def _maxk_kernel(w_ref, out_ref):
    w = w_ref[:, :]
    n = w.shape[1]
    wb = _M0 * jnp.sum(w, axis=1, keepdims=True)
    wi = jax.lax.bitcast_convert_type(w, jnp.int32)
    lo0 = jnp.full((w.shape[0], 1), -1, jnp.int32)
    hi0 = jnp.full((w.shape[0], 1), _MAX_FINITE_BITS, jnp.int32)

    def body(_, carry):
        lo, hi = carry
        mid = lo + ((hi - lo) >> 1)
        mask = wi <= mid
        s_le = jnp.sum(jnp.where(mask, w, 0.0), axis=1, keepdims=True)
        cond = s_le >= wb
        return jnp.where(cond, lo, mid), jnp.where(cond, mid, hi)

    _, hi = jax.lax.fori_loop(0, _BISECT_ITERS, body, (lo0, hi0))
    tau = jax.lax.bitcast_convert_type(hi, jnp.float32)
    mlt = wi < hi
    s_lt = jnp.sum(jnp.where(mlt, w, 0.0), axis=1, keepdims=True)
    c_lt = jnp.sum(mlt.astype(jnp.float32), axis=1, keepdims=True)
    safe_tau = jnp.where(tau > 0, tau, 1.0)
    extra = jnp.where(
        tau > 0, jnp.floor(jnp.maximum(wb - s_lt, 0.0) / safe_tau), 0.0
    )
    counts = c_lt + extra
    maxk = jnp.minimum(jnp.max(counts, axis=0, keepdims=True) + 1.0, float(n))
    out_ref[:, :] = maxk


def _squared_dists(x_ref, gt_ref):
    x = x_ref[0]
    gt = gt_ref[:, :]
    x2 = jnp.sum(x * x, axis=1, keepdims=True)
    g2 = jnp.sum(gt * gt, axis=0, keepdims=True)
    xg = jnp.dot(x, gt, preferred_element_type=jnp.float32)
    return jnp.maximum(x2 + g2 - 2.0 * xg, 0.0)


def _dtm_fast_kernel(x_ref, w_ref, gt_ref, mk_ref, out_ref, flag_ref):
    w = w_ref[0]
    mk = mk_ref[:, :]
    wb = _M0 * jnp.sum(w, axis=1, keepdims=True)
    s = _squared_dists(x_ref, gt_ref)
    si = jax.lax.bitcast_convert_type(s, jnp.int32)

    tr = s.shape[0]
    lo0 = jnp.full((tr, 1), -1, jnp.int32)
    hi0 = jnp.full((tr, 1), _MAX_FINITE_BITS, jnp.int32)

    def body(_, carry):
        lo, hi = carry
        mid = lo + ((hi - lo) >> 1)
        w_le = jnp.sum(jnp.where(si <= mid, w, 0.0), axis=1, keepdims=True)
        cond = w_le >= wb
        return jnp.where(cond, lo, mid), jnp.where(cond, mid, hi)

    _, hi = jax.lax.fori_loop(0, _BISECT_ITERS, body, (lo0, hi0))
    s_star = jax.lax.bitcast_convert_type(hi, jnp.float32)
    mlt = si < hi
    w_lt = jnp.sum(jnp.where(mlt, w, 0.0), axis=1, keepdims=True)
    g_lt = jnp.sum(jnp.where(mlt, s * w, 0.0), axis=1, keepdims=True)
    c_le = jnp.sum((si <= hi).astype(jnp.float32), axis=1, keepdims=True)
    val = g_lt + s_star * (wb - w_lt)
    out_ref[0] = jnp.sqrt(val / wb)
    flag_ref[0] = (c_le >= mk).astype(jnp.float32)


def _dtm_exact_kernel(x_ref, w_ref, gt_ref, mk_ref, out_ref):
    w = w_ref[0]
    mk = mk_ref[:, :]
    wb = _M0 * jnp.sum(w, axis=1, keepdims=True)
    s = _squared_dists(x_ref, gt_ref)
    si = jax.lax.bitcast_convert_type(s, jnp.int32)

    tr = s.shape[0]
    lo0 = jnp.full((tr, 1), -1, jnp.int32)
    hi0 = jnp.full((tr, 1), _MAX_FINITE_BITS, jnp.int32)

    def body(_, carry):
        lo, hi = carry
        mid = lo + ((hi - lo) >> 1)
        mask = si <= mid
        w_le = jnp.sum(jnp.where(mask, w, 0.0), axis=1, keepdims=True)
        c_le = jnp.sum(mask.astype(jnp.float32), axis=1, keepdims=True)
        cond = (w_le >= wb) | (c_le >= mk)
        return jnp.where(cond, lo, mid), jnp.where(cond, mid, hi)

    _, hi = jax.lax.fori_loop(0, _BISECT_ITERS, body, (lo0, hi0))
    s_star = jax.lax.bitcast_convert_type(hi, jnp.float32)
    mlt = si < hi
    w_lt = jnp.sum(jnp.where(mlt, w, 0.0), axis=1, keepdims=True)
    g_lt = jnp.sum(jnp.where(mlt, s * w, 0.0), axis=1, keepdims=True)
    val = g_lt + s_star * (wb - w_lt)
    out_ref[0] = jnp.sqrt(val / wb)


def _dtm_call(body, b, n, d, m, tr, n_outs):
    outs = [jax.ShapeDtypeStruct((b, n, 1), jnp.float32)] * n_outs
    specs = [pl.BlockSpec((1, tr, 1), lambda bi, ti: (bi, ti, 0))] * n_outs
    return pl.pallas_call(
        body,
        grid=(b, n // tr),
        in_specs=[
            pl.BlockSpec((1, tr, d), lambda bi, ti: (bi, ti, 0)),
            pl.BlockSpec((1, 1, m), lambda bi, ti: (bi, 0, 0)),
            pl.BlockSpec((d, m), lambda bi, ti: (0, 0)),
            pl.BlockSpec((1, 1), lambda bi, ti: (0, 0)),
        ],
        out_specs=specs if n_outs > 1 else specs[0],
        out_shape=outs if n_outs > 1 else outs[0],
    )


@jax.jit
def kernel(input, weight, grid):
    b, n, d = input.shape
    m = grid.shape[0]

    maxk = pl.pallas_call(
        _maxk_kernel,
        out_shape=jax.ShapeDtypeStruct((1, 1), jnp.float32),
    )(weight)

    tr = 512 if n % 512 == 0 else n
    gt = grid.T
    w3 = weight.reshape(b, 1, n)

    fast, flags = _dtm_call(_dtm_fast_kernel, b, n, d, m, tr, 2)(
        input, w3, gt, maxk
    )

    def corrected(_):
        exact = _dtm_call(_dtm_exact_kernel, b, n, d, m, tr, 1)(
            input, w3, gt, maxk
        )
        return jnp.where(flags > 0, exact, fast)

    out = jax.lax.cond(
        jnp.any(flags > 0), corrected, lambda _: fast, operand=None
    )
    return out[:, :, 0]


# speedup vs baseline: 774.1691x; 1.6574x over previous
import functools

import jax
import jax.numpy as jnp
from jax.experimental import pallas as pl

_M0 = 0.05
_MAX_FINITE_BITS = 0x7F7FFFFF
_BISECT_ITERS = 31


def _maxk_kernel(w_ref, out_ref):
    w = w_ref[:, :]
    n = w.shape[1]
    wb = _M0 * jnp.sum(w, axis=1, keepdims=True)
    wi = jax.lax.bitcast_convert_type(w, jnp.int32)
    lo0 = jnp.full((w.shape[0], 1), -1, jnp.int32)
    hi0 = jnp.full((w.shape[0], 1), _MAX_FINITE_BITS, jnp.int32)

    def body(_, carry):
        lo, hi = carry
        mid = lo + ((hi - lo) >> 1)
        mask = wi <= mid
        s_le = jnp.sum(jnp.where(mask, w, 0.0), axis=1, keepdims=True)
        cond = s_le >= wb
        return jnp.where(cond, lo, mid), jnp.where(cond, mid, hi)

    _, hi = jax.lax.fori_loop(0, _BISECT_ITERS, body, (lo0, hi0))
    tau = jax.lax.bitcast_convert_type(hi, jnp.float32)
    mlt = wi < hi
    s_lt = jnp.sum(jnp.where(mlt, w, 0.0), axis=1, keepdims=True)
    c_lt = jnp.sum(mlt.astype(jnp.float32), axis=1, keepdims=True)
    safe_tau = jnp.where(tau > 0, tau, 1.0)
    extra = jnp.where(
        tau > 0, jnp.floor(jnp.maximum(wb - s_lt, 0.0) / safe_tau), 0.0
    )
    counts = c_lt + extra
    maxk = jnp.minimum(jnp.max(counts, axis=0, keepdims=True) + 1.0, float(n))
    out_ref[:, :] = maxk


def _sq_dists_t(xt_ref, g_ref):
    xt = xt_ref[0]
    g = g_ref[:, :]
    x2 = jnp.sum(xt * xt, axis=0, keepdims=True)
    g2 = jnp.sum(g * g, axis=1, keepdims=True)
    gx = jnp.dot(g, xt, preferred_element_type=jnp.float32)
    return jnp.maximum(g2 + x2 - 2.0 * gx, 0.0)


def _dtm_fast_kernel(xt_ref, w_ref, g_ref, mk_ref, out_ref, flag_ref):
    w = w_ref[0]
    mk = mk_ref[:, :]
    wb = _M0 * jnp.sum(w, axis=0, keepdims=True)
    s = _sq_dists_t(xt_ref, g_ref)
    si = jax.lax.bitcast_convert_type(s, jnp.int32)
    wbc = jnp.broadcast_to(w, s.shape)

    tr = s.shape[1]
    lo0 = jnp.full((1, tr), -1, jnp.int32)
    hi0 = jnp.full((1, tr), _MAX_FINITE_BITS, jnp.int32)

    def body(_, carry):
        lo, hi = carry
        mid = lo + ((hi - lo) >> 1)
        w_le = jnp.sum(
            jnp.where(si <= mid, wbc, 0.0), axis=0, keepdims=True
        )
        cond = w_le >= wb
        return jnp.where(cond, lo, mid), jnp.where(cond, mid, hi)

    _, hi = jax.lax.fori_loop(0, _BISECT_ITERS, body, (lo0, hi0))
    s_star = jax.lax.bitcast_convert_type(hi, jnp.float32)
    mlt = si < hi
    w_lt = jnp.sum(jnp.where(mlt, wbc, 0.0), axis=0, keepdims=True)
    g_lt = jnp.sum(jnp.where(mlt, s * wbc, 0.0), axis=0, keepdims=True)
    c_le = jnp.sum((si <= hi).astype(jnp.float32), axis=0, keepdims=True)
    val = g_lt + s_star * (wb - w_lt)
    out_ref[0] = jnp.sqrt(val / wb)
    flag_ref[0] = (c_le >= mk).astype(jnp.float32)


def _dtm_exact_kernel(xt_ref, w_ref, g_ref, mk_ref, out_ref):
    w = w_ref[0]
    mk = mk_ref[:, :]
    wb = _M0 * jnp.sum(w, axis=0, keepdims=True)
    s = _sq_dists_t(xt_ref, g_ref)
    si = jax.lax.bitcast_convert_type(s, jnp.int32)
    wbc = jnp.broadcast_to(w, s.shape)

    tr = s.shape[1]
    lo0 = jnp.full((1, tr), -1, jnp.int32)
    hi0 = jnp.full((1, tr), _MAX_FINITE_BITS, jnp.int32)

    def body(_, carry):
        lo, hi = carry
        mid = lo + ((hi - lo) >> 1)
        mask = si <= mid
        w_le = jnp.sum(jnp.where(mask, wbc, 0.0), axis=0, keepdims=True)
        c_le = jnp.sum(mask.astype(jnp.float32), axis=0, keepdims=True)
        cond = (w_le >= wb) | (c_le >= mk)
        return jnp.where(cond, lo, mid), jnp.where(cond, mid, hi)

    _, hi = jax.lax.fori_loop(0, _BISECT_ITERS, body, (lo0, hi0))
    s_star = jax.lax.bitcast_convert_type(hi, jnp.float32)
    mlt = si < hi
    w_lt = jnp.sum(jnp.where(mlt, wbc, 0.0), axis=0, keepdims=True)
    g_lt = jnp.sum(jnp.where(mlt, s * wbc, 0.0), axis=0, keepdims=True)
    val = g_lt + s_star * (wb - w_lt)
    out_ref[0] = jnp.sqrt(val / wb)


def _dtm_call(body, b, n, d, m, tr, n_outs):
    outs = [jax.ShapeDtypeStruct((b, 1, n), jnp.float32)] * n_outs
    specs = [pl.BlockSpec((1, 1, tr), lambda bi, ti: (bi, 0, ti))] * n_outs
    return pl.pallas_call(
        body,
        grid=(b, n // tr),
        in_specs=[
            pl.BlockSpec((1, d, tr), lambda bi, ti: (bi, 0, ti)),
            pl.BlockSpec((1, m, 1), lambda bi, ti: (bi, 0, 0)),
            pl.BlockSpec((m, d), lambda bi, ti: (0, 0)),
            pl.BlockSpec((1, 1), lambda bi, ti: (0, 0)),
        ],
        out_specs=specs if n_outs > 1 else specs[0],
        out_shape=outs if n_outs > 1 else outs[0],
    )


@jax.jit
def kernel(input, weight, grid):
    b, n, d = input.shape
    m = grid.shape[0]

    maxk = pl.pallas_call(
        _maxk_kernel,
        out_shape=jax.ShapeDtypeStruct((1, 1), jnp.float32),
    )(weight)

    tr = 512 if n % 512 == 0 else n
    xt = input.transpose(0, 2, 1)
    w3 = weight.reshape(b, n, 1)

    fast, flags = _dtm_call(_dtm_fast_kernel, b, n, d, m, tr, 2)(
        xt, w3, grid, maxk
    )

    def corrected(_):
        exact = _dtm_call(_dtm_exact_kernel, b, n, d, m, tr, 1)(
            xt, w3, grid, maxk
        )
        return jnp.where(flags > 0, exact, fast)

    out = jax.lax.cond(
        jnp.any(flags > 0), corrected, lambda _: fast, operand=None
    )
    return out[:, 0, :]
